# baseline (speedup 1.0000x reference)
import functools

import jax
import jax.numpy as jnp
from jax import lax
from jax.experimental import pallas as pl
from jax.experimental.pallas import tpu as pltpu
from jax.experimental.pallas import tpu_sc as plsc

NBR = 128
ANG = 64
E = 160000
A = 320000
NC = 256
INV_SQRT_2 = 1.0 / 2.0 ** 0.5

BLK = 2560
GRID = A // BLK

SC_CORES = 2
SC_SUBCORES = 16
NW = SC_CORES * SC_SUBCORES
PER_W = A // NW
CHUNK = 80
N_CHUNKS = PER_W // CHUNK


def _sc_gather_add(p0, p1, idx0, idx1):
    mesh = plsc.VectorSubcoreMesh(
        core_axis_name="c", subcore_axis_name="s",
        num_cores=SC_CORES, num_subcores=SC_SUBCORES)

    @functools.partial(
        pl.kernel,
        out_type=jax.ShapeDtypeStruct((A, NBR), jnp.float32),
        mesh=mesh,
        scratch_types=[
            pltpu.VMEM((3, CHUNK), jnp.int32),
            pltpu.VMEM((3, CHUNK), jnp.int32),
            pltpu.VMEM((3, CHUNK, NBR), jnp.float32),
            pltpu.SemaphoreType.DMA((3,)),
            pltpu.SemaphoreType.DMA((3,)),
            pltpu.SemaphoreType.DMA((3,)),
        ],
    )
    def gather_kernel(p0_hbm, p1_hbm, i0_hbm, i1_hbm, out_hbm,
                      i0_v, i1_v, rows_v, sem_i, sem_g, sem_o):
        wid = lax.axis_index("s") * SC_CORES + lax.axis_index("c")
        base = wid * PER_W

        def off_of(j):
            return pl.multiple_of(base + j * CHUNK, 8)

        def issue_idx(j, k):
            off = off_of(j)
            pltpu.async_copy(i0_hbm.at[pl.ds(off, CHUNK)], i0_v.at[k],
                             sem_i.at[k])
            pltpu.async_copy(i1_hbm.at[pl.ds(off, CHUNK)], i1_v.at[k],
                             sem_i.at[k])

        def wait_idx(j, k):
            off = off_of(j)
            pltpu.make_async_copy(i0_hbm.at[pl.ds(off, CHUNK)], i0_v.at[k],
                                  sem_i.at[k]).wait()
            pltpu.make_async_copy(i1_hbm.at[pl.ds(off, CHUNK)], i1_v.at[k],
                                  sem_i.at[k]).wait()

        def wait_out(j, k):
            off = off_of(j)
            pltpu.make_async_copy(rows_v.at[k], out_hbm.at[pl.ds(off, CHUNK)],
                                  sem_o.at[k]).wait()

        issue_idx(0, 0)

        def body(j, carry):
            k = lax.rem(j, 3)

            @pl.when(j < N_CHUNKS)
            def _gather():
                @pl.when(j >= 3)
                def _slot_free():
                    wait_out(j - 3, k)
                wait_idx(j, k)
                pltpu.async_copy(p0_hbm.at[i0_v.at[k]], rows_v.at[k],
                                 sem_g.at[k])

            @pl.when(j + 1 < N_CHUNKS)
            def _prefetch_idx():
                issue_idx(j + 1, lax.rem(j + 1, 3))

            @pl.when(jnp.logical_and(j >= 1, j - 1 < N_CHUNKS))
            def _add():
                k1 = lax.rem(j - 1, 3)
                pltpu.make_async_copy(p0_hbm.at[i0_v.at[k1]], rows_v.at[k1],
                                      sem_g.at[k1]).wait()
                pltpu.async_copy(p1_hbm.at[i1_v.at[k1]], rows_v.at[k1],
                                 sem_g.at[k1], add=True)

            @pl.when(jnp.logical_and(j >= 2, j - 2 < N_CHUNKS))
            def _writeback():
                k2 = lax.rem(j - 2, 3)
                pltpu.make_async_copy(p1_hbm.at[i1_v.at[k2]], rows_v.at[k2],
                                      sem_g.at[k2]).wait()
                pltpu.async_copy(rows_v.at[k2], out_hbm.at[
                    pl.ds(off_of(j - 2), CHUNK)], sem_o.at[k2])

            return carry

        lax.fori_loop(0, N_CHUNKS + 2, body, 0)
        for jj in range(N_CHUNKS - 3, N_CHUNKS):
            wait_out(jj, jj % 3)

    return gather_kernel(p0, p1, idx0, idx1)


BLKE = 640
GRID_E = E // BLKE


def _pre_body(edge_ref, w0_ref, w1_ref, p0_ref, p1_ref):
    e = edge_ref[...]
    p0_ref[...] = _dot(e, w0_ref[...])
    p1_ref[...] = _dot(e, w1_ref[...])


def _onehot_t(idx):
    return (lax.broadcasted_iota(jnp.int32, (NC, BLK), 0)
            == idx[None, :]).astype(jnp.bfloat16)


def _onehot(idx):
    return (lax.broadcasted_iota(jnp.int32, (BLK, NC), 1)
            == idx[:, None]).astype(jnp.bfloat16)


def _bf(x):
    return x.astype(jnp.bfloat16)


def _hilo(x):
    hi = x.astype(jnp.bfloat16)
    lo = (x - hi.astype(jnp.float32)).astype(jnp.bfloat16)
    return hi, lo


def _dot(a, b):
    return jnp.dot(a, b, preferred_element_type=jnp.float32)


def _dot_t(a, b, ca, cb):
    return lax.dot_general(a, b, (((ca,), (cb,)), ((), ())),
                           preferred_element_type=jnp.float32)


def _p1_body(anglet_ref, g_ref, wa_ref, idx_ref, stats_ref):
    g = g_ref[...] + _dot_t(anglet_ref[...], wa_ref[...], 0, 0)
    idx = idx_ref[0, 0, :]
    oh_t = _onehot_t(idx)

    @pl.when(pl.program_id(0) == 0)
    def _init():
        stats_ref[...] = jnp.zeros_like(stats_ref)

    stats_ref[:, :2 * ANG] += _dot(oh_t, _bf(g))
    stats_ref[:, 2 * ANG:] += _dot(oh_t, _bf(g * g))


def _p2_body(anglet_ref, g_ref, wa_ref, idx_ref, ab1h_ref, ab1l_ref,
             wm_ref, eye_ref, sumedt_ref, stats2_ref):
    g = g_ref[...] + _dot_t(anglet_ref[...], wa_ref[...], 0, 0)
    idx = idx_ref[0, 0, :]
    oh = _onehot(idx)
    gath = _dot(oh, ab1h_ref[...]) + _dot(oh, ab1l_ref[...])
    xn = g * gath[:, :2 * ANG] + gath[:, 2 * ANG:]
    core = jnp.maximum(xn[:, :ANG], 0.0)
    filt = xn[:, ANG:]
    t = jnp.tanh(_dot(filt, wm_ref[...]))
    sumed = t * core
    sumedt_ref[...] = _dot_t(eye_ref[...], sumed, 1, 1)
    oh_t = _onehot_t(idx)

    @pl.when(pl.program_id(0) == 0)
    def _init():
        stats2_ref[...] = jnp.zeros_like(stats2_ref)

    stats2_ref[:, :ANG] += _dot(oh_t, _bf(sumed))
    stats2_ref[:, ANG:] += _dot(oh_t, _bf(sumed * sumed))


def _p3_body(sumedt_ref, anglet_ref, idx_ref, ab2h_ref, ab2l_ref,
             w10_ref, b10_ref, w20_ref, b20_ref,
             w11_ref, b11_ref, w21_ref, b21_ref, outt_ref):
    idx = idx_ref[0, 0, :]
    oh_t = _onehot_t(idx)
    gath = (_dot_t(ab2h_ref[...], oh_t, 0, 0)
            + _dot_t(ab2l_ref[...], oh_t, 0, 0))
    x = sumedt_ref[...] * gath[:ANG, :] + gath[ANG:, :]
    for w1, b1, w2, b2 in ((w10_ref, b10_ref, w20_ref, b20_ref),
                           (w11_ref, b11_ref, w21_ref, b21_ref)):
        h = jnp.maximum(_dot_t(w1[...], x, 0, 0) + b1[...], 0.0)
        h = jnp.maximum(_dot_t(w2[...], h, 0, 0) + b2[...], 0.0)
        x = x + h
    outt_ref[...] = INV_SQRT_2 * jnp.maximum(anglet_ref[...] + x, 0.0)


def _finalize(stats, cnt, gamma, beta, width):
    s1, s2 = stats[:, :width], stats[:, width:]
    mean = s1 / cnt[:, None]
    var = jnp.maximum(s2 / cnt[:, None] - mean * mean, 0.0)
    a = gamma[None, :] * lax.rsqrt(var + 1e-5)
    b = beta[None, :] - a * mean
    return jnp.concatenate([a, b], axis=1)


def kernel(edge, angle, angle_nbr_idx, crystal_angle_idx, W_full, W_mask,
           gamma1, beta1, gamma2, beta2,
           res_W1_0, res_b1_0, res_W2_0, res_b2_0,
           res_W1_1, res_b1_1, res_W2_1, res_b2_1):
    f32 = jnp.float32

    wa = W_full[:ANG, :]
    w0 = W_full[ANG:ANG + NBR, :]
    w1 = W_full[ANG + NBR:, :]
    p0, p1 = pl.pallas_call(
        _pre_body,
        grid=(GRID_E,),
        in_specs=[pl.BlockSpec((BLKE, NBR), lambda i: (i, 0)),
                  pl.BlockSpec((NBR, 2 * ANG), lambda i: (0, 0)),
                  pl.BlockSpec((NBR, 2 * ANG), lambda i: (0, 0))],
        out_specs=[pl.BlockSpec((BLKE, 2 * ANG), lambda i: (i, 0)),
                   pl.BlockSpec((BLKE, 2 * ANG), lambda i: (i, 0))],
        out_shape=[jax.ShapeDtypeStruct((E, 2 * ANG), f32),
                   jax.ShapeDtypeStruct((E, 2 * ANG), f32)],
    )(edge, w0, w1)

    idx0 = angle_nbr_idx[:, 0]
    idx1 = angle_nbr_idx[:, 1]
    g_sum = _sc_gather_add(p0, p1, idx0, idx1)

    idx3 = crystal_angle_idx.reshape(GRID, 1, BLK)

    angle_t = angle.T

    bounds = jnp.searchsorted(crystal_angle_idx,
                              jnp.arange(NC + 1, dtype=jnp.int32), side="left")
    cnt = jnp.maximum((bounds[1:] - bounds[:-1]).astype(f32), 1.0)

    row_spec = lambda w: pl.BlockSpec((BLK, w), lambda i: (i, 0))
    col_spec = lambda r: pl.BlockSpec((r, BLK), lambda i: (0, i))
    full_spec = lambda r, c: pl.BlockSpec((r, c), lambda i: (0, 0))
    idx_spec = pl.BlockSpec((1, 1, BLK), lambda i: (i, 0, 0))

    stats1 = pl.pallas_call(
        _p1_body,
        grid=(GRID,),
        in_specs=[col_spec(ANG), row_spec(2 * ANG),
                  full_spec(ANG, 2 * ANG), idx_spec],
        out_specs=full_spec(NC, 4 * ANG),
        out_shape=jax.ShapeDtypeStruct((NC, 4 * ANG), f32),
    )(angle_t, g_sum, wa, idx3)

    ab1h, ab1l = _hilo(_finalize(stats1, cnt, gamma1, beta1, 2 * ANG))
    wm = jnp.tile(W_mask, (1, ANG))

    eye = jnp.eye(ANG, dtype=f32)
    sumed_t, stats2 = pl.pallas_call(
        _p2_body,
        grid=(GRID,),
        in_specs=[col_spec(ANG), row_spec(2 * ANG),
                  full_spec(ANG, 2 * ANG), idx_spec,
                  full_spec(NC, 4 * ANG), full_spec(NC, 4 * ANG),
                  full_spec(ANG, ANG), full_spec(ANG, ANG)],
        out_specs=[col_spec(ANG), full_spec(NC, 2 * ANG)],
        out_shape=[jax.ShapeDtypeStruct((ANG, A), f32),
                   jax.ShapeDtypeStruct((NC, 2 * ANG), f32)],
    )(angle_t, g_sum, wa, idx3, ab1h, ab1l, wm, eye)

    ab2h, ab2l = _hilo(_finalize(stats2, cnt, gamma2, beta2, ANG))

    mid = ANG // 2
    w10 = jnp.zeros((ANG, 2 * ANG), f32).at[:, :mid].set(res_W1_0)
    b10 = jnp.zeros((2 * ANG, 1), f32).at[:mid, 0].set(res_b1_0)
    w20 = jnp.zeros((2 * ANG, ANG), f32).at[:mid, :].set(res_W2_0)
    b20 = res_b2_0.reshape(ANG, 1)
    w11 = jnp.zeros((ANG, 2 * ANG), f32).at[:, :mid].set(res_W1_1)
    b11 = jnp.zeros((2 * ANG, 1), f32).at[:mid, 0].set(res_b1_1)
    w21 = jnp.zeros((2 * ANG, ANG), f32).at[:mid, :].set(res_W2_1)
    b21 = res_b2_1.reshape(ANG, 1)

    out_t = pl.pallas_call(
        _p3_body,
        grid=(GRID,),
        in_specs=[col_spec(ANG), col_spec(ANG), idx_spec,
                  full_spec(NC, 2 * ANG), full_spec(NC, 2 * ANG),
                  full_spec(ANG, 2 * ANG), full_spec(2 * ANG, 1),
                  full_spec(2 * ANG, ANG), full_spec(ANG, 1),
                  full_spec(ANG, 2 * ANG), full_spec(2 * ANG, 1),
                  full_spec(2 * ANG, ANG), full_spec(ANG, 1)],
        out_specs=col_spec(ANG),
        out_shape=jax.ShapeDtypeStruct((ANG, A), f32),
    )(sumed_t, angle_t, idx3, ab2h, ab2l,
      w10, b10, w20, b20, w11, b11, w21, b21)

    return out_t.T


# speedup vs baseline: 1.0247x; 1.0247x over previous
import functools

import jax
import jax.numpy as jnp
from jax import lax
from jax.experimental import pallas as pl
from jax.experimental.pallas import tpu as pltpu
from jax.experimental.pallas import tpu_sc as plsc

NBR = 128
ANG = 64
E = 160000
A = 320000
NC = 256
INV_SQRT_2 = 1.0 / 2.0 ** 0.5

BLK = 2560
GRID = A // BLK

SC_CORES = 2
SC_SUBCORES = 16
NW = SC_CORES * SC_SUBCORES
PER_W = A // NW
CHUNK = 80
N_CHUNKS = PER_W // CHUNK


def _sc_gather_add(p0, p1, idx0, idx1):
    mesh = plsc.VectorSubcoreMesh(
        core_axis_name="c", subcore_axis_name="s",
        num_cores=SC_CORES, num_subcores=SC_SUBCORES)

    @functools.partial(
        pl.kernel,
        out_type=jax.ShapeDtypeStruct((A, NBR), jnp.float32),
        mesh=mesh,
        scratch_types=[
            pltpu.VMEM((3, CHUNK), jnp.int32),
            pltpu.VMEM((3, CHUNK), jnp.int32),
            pltpu.VMEM((3, CHUNK, NBR), jnp.float32),
            pltpu.SemaphoreType.DMA((3,)),
            pltpu.SemaphoreType.DMA((3,)),
            pltpu.SemaphoreType.DMA((3,)),
        ],
    )
    def gather_kernel(p0_hbm, p1_hbm, i0_hbm, i1_hbm, out_hbm,
                      i0_v, i1_v, rows_v, sem_i, sem_g, sem_o):
        wid = lax.axis_index("s") * SC_CORES + lax.axis_index("c")
        base = wid * PER_W

        def off_of(j):
            return pl.multiple_of(base + j * CHUNK, 8)

        def issue_idx(j, k):
            off = off_of(j)
            pltpu.async_copy(i0_hbm.at[pl.ds(off, CHUNK)], i0_v.at[k],
                             sem_i.at[k])
            pltpu.async_copy(i1_hbm.at[pl.ds(off, CHUNK)], i1_v.at[k],
                             sem_i.at[k])

        def wait_idx(j, k):
            off = off_of(j)
            pltpu.make_async_copy(i0_hbm.at[pl.ds(off, CHUNK)], i0_v.at[k],
                                  sem_i.at[k]).wait()
            pltpu.make_async_copy(i1_hbm.at[pl.ds(off, CHUNK)], i1_v.at[k],
                                  sem_i.at[k]).wait()

        def wait_out(j, k):
            off = off_of(j)
            pltpu.make_async_copy(rows_v.at[k], out_hbm.at[pl.ds(off, CHUNK)],
                                  sem_o.at[k]).wait()

        issue_idx(0, 0)

        def body(j, carry):
            k = lax.rem(j, 3)

            @pl.when(j < N_CHUNKS)
            def _gather():
                @pl.when(j >= 3)
                def _slot_free():
                    wait_out(j - 3, k)
                wait_idx(j, k)
                pltpu.async_copy(p0_hbm.at[i0_v.at[k]], rows_v.at[k],
                                 sem_g.at[k])

            @pl.when(j + 1 < N_CHUNKS)
            def _prefetch_idx():
                issue_idx(j + 1, lax.rem(j + 1, 3))

            @pl.when(jnp.logical_and(j >= 1, j - 1 < N_CHUNKS))
            def _add():
                k1 = lax.rem(j - 1, 3)
                pltpu.make_async_copy(p0_hbm.at[i0_v.at[k1]], rows_v.at[k1],
                                      sem_g.at[k1]).wait()
                pltpu.async_copy(p1_hbm.at[i1_v.at[k1]], rows_v.at[k1],
                                 sem_g.at[k1], add=True)

            @pl.when(jnp.logical_and(j >= 2, j - 2 < N_CHUNKS))
            def _writeback():
                k2 = lax.rem(j - 2, 3)
                pltpu.make_async_copy(p1_hbm.at[i1_v.at[k2]], rows_v.at[k2],
                                      sem_g.at[k2]).wait()
                pltpu.async_copy(rows_v.at[k2], out_hbm.at[
                    pl.ds(off_of(j - 2), CHUNK)], sem_o.at[k2])

            return carry

        lax.fori_loop(0, N_CHUNKS + 2, body, 0)
        for jj in range(N_CHUNKS - 3, N_CHUNKS):
            wait_out(jj, jj % 3)

    return gather_kernel(p0, p1, idx0, idx1)


BLKE = 640
GRID_E = E // BLKE


def _pre_body(edge_ref, w0_ref, w1_ref, p0_ref, p1_ref):
    e = edge_ref[...]
    p0_ref[...] = _dot(e, w0_ref[...])
    p1_ref[...] = _dot(e, w1_ref[...])


def _onehot_t(idx, dt=jnp.float32):
    return (lax.broadcasted_iota(jnp.int32, (NC, BLK), 0)
            == idx[None, :]).astype(dt)


def _onehot(idx, dt=jnp.float32):
    return (lax.broadcasted_iota(jnp.int32, (BLK, NC), 1)
            == idx[:, None]).astype(dt)


def _bf(x):
    return x.astype(jnp.bfloat16)


def _hilo(x):
    hi = x.astype(jnp.bfloat16)
    lo = (x - hi.astype(jnp.float32)).astype(jnp.bfloat16)
    return hi, lo


def _dot(a, b):
    return jnp.dot(a, b, preferred_element_type=jnp.float32)


def _dot_t(a, b, ca, cb):
    return lax.dot_general(a, b, (((ca,), (cb,)), ((), ())),
                           preferred_element_type=jnp.float32)


def _p1_body(anglet_ref, g_ref, wa_ref, idx_ref, stats_ref):
    g = g_ref[...] + _dot_t(anglet_ref[...], wa_ref[...], 0, 0)
    idx = idx_ref[0, 0, :]
    oh_t = _onehot_t(idx)

    @pl.when(pl.program_id(0) == 0)
    def _init():
        stats_ref[...] = jnp.zeros_like(stats_ref)

    stats_ref[:, :2 * ANG] += _dot(oh_t, g)
    stats_ref[:, 2 * ANG:] += _dot(oh_t, g * g)


def _p2_body(anglet_ref, g_ref, wa_ref, idx_ref, ab1h_ref, ab1l_ref,
             wm_ref, eye_ref, sumedt_ref, stats2_ref):
    g = g_ref[...] + _dot_t(anglet_ref[...], wa_ref[...], 0, 0)
    idx = idx_ref[0, 0, :]
    oh = _onehot(idx, jnp.bfloat16)
    gath = _dot(oh, ab1h_ref[...]) + _dot(oh, ab1l_ref[...])
    xn = g * gath[:, :2 * ANG] + gath[:, 2 * ANG:]
    core = jnp.maximum(xn[:, :ANG], 0.0)
    filt = xn[:, ANG:]
    t = jnp.tanh(_dot(filt, wm_ref[...]))
    sumed = t * core
    sumedt_ref[...] = _dot_t(eye_ref[...], sumed, 1, 1)
    oh_t = _onehot_t(idx)

    @pl.when(pl.program_id(0) == 0)
    def _init():
        stats2_ref[...] = jnp.zeros_like(stats2_ref)

    stats2_ref[:, :ANG] += _dot(oh_t, sumed)
    stats2_ref[:, ANG:] += _dot(oh_t, sumed * sumed)


def _p3_body(sumedt_ref, anglet_ref, idx_ref, ab2h_ref, ab2l_ref,
             w10_ref, b10_ref, w20_ref, b20_ref,
             w11_ref, b11_ref, w21_ref, b21_ref, outt_ref):
    idx = idx_ref[0, 0, :]
    oh_t = _onehot_t(idx, jnp.bfloat16)
    gath = (_dot_t(ab2h_ref[...], oh_t, 0, 0)
            + _dot_t(ab2l_ref[...], oh_t, 0, 0))
    x = sumedt_ref[...] * gath[:ANG, :] + gath[ANG:, :]
    for w1, b1, w2, b2 in ((w10_ref, b10_ref, w20_ref, b20_ref),
                           (w11_ref, b11_ref, w21_ref, b21_ref)):
        h = jnp.maximum(_dot_t(w1[...], x, 0, 0) + b1[...], 0.0)
        h = jnp.maximum(_dot_t(w2[...], h, 0, 0) + b2[...], 0.0)
        x = x + h
    outt_ref[...] = INV_SQRT_2 * jnp.maximum(anglet_ref[...] + x, 0.0)


def _finalize(stats, cnt, gamma, beta, width):
    s1, s2 = stats[:, :width], stats[:, width:]
    mean = s1 / cnt[:, None]
    var = jnp.maximum(s2 / cnt[:, None] - mean * mean, 0.0)
    a = gamma[None, :] * lax.rsqrt(var + 1e-5)
    b = beta[None, :] - a * mean
    return jnp.concatenate([a, b], axis=1)


def kernel(edge, angle, angle_nbr_idx, crystal_angle_idx, W_full, W_mask,
           gamma1, beta1, gamma2, beta2,
           res_W1_0, res_b1_0, res_W2_0, res_b2_0,
           res_W1_1, res_b1_1, res_W2_1, res_b2_1):
    f32 = jnp.float32

    wa = W_full[:ANG, :]
    w0 = W_full[ANG:ANG + NBR, :]
    w1 = W_full[ANG + NBR:, :]
    p0, p1 = pl.pallas_call(
        _pre_body,
        grid=(GRID_E,),
        in_specs=[pl.BlockSpec((BLKE, NBR), lambda i: (i, 0)),
                  pl.BlockSpec((NBR, 2 * ANG), lambda i: (0, 0)),
                  pl.BlockSpec((NBR, 2 * ANG), lambda i: (0, 0))],
        out_specs=[pl.BlockSpec((BLKE, 2 * ANG), lambda i: (i, 0)),
                   pl.BlockSpec((BLKE, 2 * ANG), lambda i: (i, 0))],
        out_shape=[jax.ShapeDtypeStruct((E, 2 * ANG), f32),
                   jax.ShapeDtypeStruct((E, 2 * ANG), f32)],
    )(edge, w0, w1)

    idx0 = angle_nbr_idx[:, 0]
    idx1 = angle_nbr_idx[:, 1]
    g_sum = _sc_gather_add(p0, p1, idx0, idx1)

    idx3 = crystal_angle_idx.reshape(GRID, 1, BLK)

    angle_t = angle.T

    bounds = jnp.searchsorted(crystal_angle_idx,
                              jnp.arange(NC + 1, dtype=jnp.int32), side="left")
    cnt = jnp.maximum((bounds[1:] - bounds[:-1]).astype(f32), 1.0)

    row_spec = lambda w: pl.BlockSpec((BLK, w), lambda i: (i, 0))
    col_spec = lambda r: pl.BlockSpec((r, BLK), lambda i: (0, i))
    full_spec = lambda r, c: pl.BlockSpec((r, c), lambda i: (0, 0))
    idx_spec = pl.BlockSpec((1, 1, BLK), lambda i: (i, 0, 0))

    stats1 = pl.pallas_call(
        _p1_body,
        grid=(GRID,),
        in_specs=[col_spec(ANG), row_spec(2 * ANG),
                  full_spec(ANG, 2 * ANG), idx_spec],
        out_specs=full_spec(NC, 4 * ANG),
        out_shape=jax.ShapeDtypeStruct((NC, 4 * ANG), f32),
    )(angle_t, g_sum, wa, idx3)

    ab1h, ab1l = _hilo(_finalize(stats1, cnt, gamma1, beta1, 2 * ANG))
    wm = jnp.tile(W_mask, (1, ANG))

    eye = jnp.eye(ANG, dtype=f32)
    sumed_t, stats2 = pl.pallas_call(
        _p2_body,
        grid=(GRID,),
        in_specs=[col_spec(ANG), row_spec(2 * ANG),
                  full_spec(ANG, 2 * ANG), idx_spec,
                  full_spec(NC, 4 * ANG), full_spec(NC, 4 * ANG),
                  full_spec(ANG, ANG), full_spec(ANG, ANG)],
        out_specs=[col_spec(ANG), full_spec(NC, 2 * ANG)],
        out_shape=[jax.ShapeDtypeStruct((ANG, A), f32),
                   jax.ShapeDtypeStruct((NC, 2 * ANG), f32)],
    )(angle_t, g_sum, wa, idx3, ab1h, ab1l, wm, eye)

    ab2h, ab2l = _hilo(_finalize(stats2, cnt, gamma2, beta2, ANG))

    mid = ANG // 2
    w10 = jnp.zeros((ANG, 2 * ANG), f32).at[:, :mid].set(res_W1_0)
    b10 = jnp.zeros((2 * ANG, 1), f32).at[:mid, 0].set(res_b1_0)
    w20 = jnp.zeros((2 * ANG, ANG), f32).at[:mid, :].set(res_W2_0)
    b20 = res_b2_0.reshape(ANG, 1)
    w11 = jnp.zeros((ANG, 2 * ANG), f32).at[:, :mid].set(res_W1_1)
    b11 = jnp.zeros((2 * ANG, 1), f32).at[:mid, 0].set(res_b1_1)
    w21 = jnp.zeros((2 * ANG, ANG), f32).at[:mid, :].set(res_W2_1)
    b21 = res_b2_1.reshape(ANG, 1)

    out_t = pl.pallas_call(
        _p3_body,
        grid=(GRID,),
        in_specs=[col_spec(ANG), col_spec(ANG), idx_spec,
                  full_spec(NC, 2 * ANG), full_spec(NC, 2 * ANG),
                  full_spec(ANG, 2 * ANG), full_spec(2 * ANG, 1),
                  full_spec(2 * ANG, ANG), full_spec(ANG, 1),
                  full_spec(ANG, 2 * ANG), full_spec(2 * ANG, 1),
                  full_spec(2 * ANG, ANG), full_spec(ANG, 1)],
        out_specs=col_spec(ANG),
        out_shape=jax.ShapeDtypeStruct((ANG, A), f32),
    )(sumed_t, angle_t, idx3, ab2h, ab2l,
      w10, b10, w20, b20, w11, b11, w21, b21)

    return out_t.T


# speedup vs baseline: 1.0452x; 1.0200x over previous
import functools

import jax
import jax.numpy as jnp
from jax import lax
from jax.experimental import pallas as pl
from jax.experimental.pallas import tpu as pltpu
from jax.experimental.pallas import tpu_sc as plsc

NBR = 128
ANG = 64
E = 160000
A = 320000
NC = 256
INV_SQRT_2 = 1.0 / 2.0 ** 0.5

BLK = 2560
GRID = A // BLK

SC_CORES = 2
SC_SUBCORES = 16
NW = SC_CORES * SC_SUBCORES
PER_W = A // NW
CHUNK = 80
N_CHUNKS = PER_W // CHUNK


def _sc_gather_add(p0, p1, idx0, idx1):
    mesh = plsc.VectorSubcoreMesh(
        core_axis_name="c", subcore_axis_name="s",
        num_cores=SC_CORES, num_subcores=SC_SUBCORES)

    @functools.partial(
        pl.kernel,
        out_type=jax.ShapeDtypeStruct((A, NBR), jnp.float32),
        mesh=mesh,
        scratch_types=[
            pltpu.VMEM((3, CHUNK), jnp.int32),
            pltpu.VMEM((3, CHUNK), jnp.int32),
            pltpu.VMEM((3, CHUNK, NBR), jnp.float32),
            pltpu.SemaphoreType.DMA((3,)),
            pltpu.SemaphoreType.DMA((3,)),
            pltpu.SemaphoreType.DMA((3,)),
        ],
    )
    def gather_kernel(p0_hbm, p1_hbm, i0_hbm, i1_hbm, out_hbm,
                      i0_v, i1_v, rows_v, sem_i, sem_g, sem_o):
        wid = lax.axis_index("s") * SC_CORES + lax.axis_index("c")
        base = wid * PER_W

        def off_of(j):
            return pl.multiple_of(base + j * CHUNK, 8)

        def issue_idx(j, k):
            off = off_of(j)
            pltpu.async_copy(i0_hbm.at[pl.ds(off, CHUNK)], i0_v.at[k],
                             sem_i.at[k])
            pltpu.async_copy(i1_hbm.at[pl.ds(off, CHUNK)], i1_v.at[k],
                             sem_i.at[k])

        def wait_idx(j, k):
            off = off_of(j)
            pltpu.make_async_copy(i0_hbm.at[pl.ds(off, CHUNK)], i0_v.at[k],
                                  sem_i.at[k]).wait()
            pltpu.make_async_copy(i1_hbm.at[pl.ds(off, CHUNK)], i1_v.at[k],
                                  sem_i.at[k]).wait()

        def wait_out(j, k):
            off = off_of(j)
            pltpu.make_async_copy(rows_v.at[k], out_hbm.at[pl.ds(off, CHUNK)],
                                  sem_o.at[k]).wait()

        issue_idx(0, 0)

        def body(j, carry):
            k = lax.rem(j, 3)

            @pl.when(j < N_CHUNKS)
            def _gather():
                @pl.when(j >= 3)
                def _slot_free():
                    wait_out(j - 3, k)
                wait_idx(j, k)
                pltpu.async_copy(p0_hbm.at[i0_v.at[k]], rows_v.at[k],
                                 sem_g.at[k])

            @pl.when(j + 1 < N_CHUNKS)
            def _prefetch_idx():
                issue_idx(j + 1, lax.rem(j + 1, 3))

            @pl.when(jnp.logical_and(j >= 1, j - 1 < N_CHUNKS))
            def _add():
                k1 = lax.rem(j - 1, 3)
                pltpu.make_async_copy(p0_hbm.at[i0_v.at[k1]], rows_v.at[k1],
                                      sem_g.at[k1]).wait()
                pltpu.async_copy(p1_hbm.at[i1_v.at[k1]], rows_v.at[k1],
                                 sem_g.at[k1], add=True)

            @pl.when(jnp.logical_and(j >= 2, j - 2 < N_CHUNKS))
            def _writeback():
                k2 = lax.rem(j - 2, 3)
                pltpu.make_async_copy(p1_hbm.at[i1_v.at[k2]], rows_v.at[k2],
                                      sem_g.at[k2]).wait()
                pltpu.async_copy(rows_v.at[k2], out_hbm.at[
                    pl.ds(off_of(j - 2), CHUNK)], sem_o.at[k2])

            return carry

        lax.fori_loop(0, N_CHUNKS + 2, body, 0)
        for jj in range(N_CHUNKS - 3, N_CHUNKS):
            wait_out(jj, jj % 3)

    return gather_kernel(p0, p1, idx0, idx1)


BLKE = 640
GRID_E = E // BLKE


def _pre_body(edge_ref, w0_ref, w1_ref, p0_ref, p1_ref):
    e = edge_ref[...]
    p0_ref[...] = _dot(e, w0_ref[...])
    p1_ref[...] = _dot(e, w1_ref[...])


def _onehot_t(idx, dt=jnp.float32):
    return (lax.broadcasted_iota(jnp.int32, (NC, BLK), 0)
            == idx[None, :]).astype(dt)


def _onehot(idx, dt=jnp.float32):
    return (lax.broadcasted_iota(jnp.int32, (BLK, NC), 1)
            == idx[:, None]).astype(dt)


def _dot(a, b):
    return jnp.dot(a, b, preferred_element_type=jnp.float32)


def _dot_t(a, b, ca, cb):
    return lax.dot_general(a, b, (((ca,), (cb,)), ((), ())),
                           preferred_element_type=jnp.float32)


def _p1_body(anglet_ref, g_ref, wa_ref, idx_ref, stats_ref):
    g = g_ref[...] + _dot_t(anglet_ref[...], wa_ref[...], 0, 0)
    idx = idx_ref[0, 0, :]
    oh_t = _onehot_t(idx)

    @pl.when(pl.program_id(0) == 0)
    def _init():
        stats_ref[...] = jnp.zeros_like(stats_ref)

    stats_ref[:, :2 * ANG] += _dot(oh_t, g)
    stats_ref[:, 2 * ANG:] += _dot(oh_t, g * g)


def _p2_body(anglet_ref, g_ref, wa_ref, idx_ref, ab1_ref,
             wm_ref, eye_ref, sumedt_ref, stats2_ref):
    g = g_ref[...] + _dot_t(anglet_ref[...], wa_ref[...], 0, 0)
    idx = idx_ref[0, 0, :]
    oh = _onehot(idx)
    gath = _dot(oh, ab1_ref[...])
    xn = g * gath[:, :2 * ANG] + gath[:, 2 * ANG:]
    core = jnp.maximum(xn[:, :ANG], 0.0)
    filt = xn[:, ANG:]
    t = jnp.tanh(_dot(filt, wm_ref[...]))
    sumed = t * core
    sumedt_ref[...] = _dot_t(eye_ref[...], sumed, 1, 1)
    oh_t = _onehot_t(idx)

    @pl.when(pl.program_id(0) == 0)
    def _init():
        stats2_ref[...] = jnp.zeros_like(stats2_ref)

    stats2_ref[:, :ANG] += _dot(oh_t, sumed)
    stats2_ref[:, ANG:] += _dot(oh_t, sumed * sumed)


def _p3_body(sumedt_ref, anglet_ref, idx_ref, ab2_ref,
             w10_ref, b10_ref, w20_ref, b20_ref,
             w11_ref, b11_ref, w21_ref, b21_ref, outt_ref):
    idx = idx_ref[0, 0, :]
    oh_t = _onehot_t(idx)
    gath = _dot_t(ab2_ref[...], oh_t, 0, 0)
    x = sumedt_ref[...] * gath[:ANG, :] + gath[ANG:, :]
    for w1, b1, w2, b2 in ((w10_ref, b10_ref, w20_ref, b20_ref),
                           (w11_ref, b11_ref, w21_ref, b21_ref)):
        h = jnp.maximum(_dot_t(w1[...], x, 0, 0) + b1[...], 0.0)
        h = jnp.maximum(_dot_t(w2[...], h, 0, 0) + b2[...], 0.0)
        x = x + h
    outt_ref[...] = INV_SQRT_2 * jnp.maximum(anglet_ref[...] + x, 0.0)


def _finalize(stats, cnt, gamma, beta, width):
    s1, s2 = stats[:, :width], stats[:, width:]
    mean = s1 / cnt[:, None]
    var = jnp.maximum(s2 / cnt[:, None] - mean * mean, 0.0)
    a = gamma[None, :] * lax.rsqrt(var + 1e-5)
    b = beta[None, :] - a * mean
    return jnp.concatenate([a, b], axis=1)


def kernel(edge, angle, angle_nbr_idx, crystal_angle_idx, W_full, W_mask,
           gamma1, beta1, gamma2, beta2,
           res_W1_0, res_b1_0, res_W2_0, res_b2_0,
           res_W1_1, res_b1_1, res_W2_1, res_b2_1):
    f32 = jnp.float32

    wa = W_full[:ANG, :]
    w0 = W_full[ANG:ANG + NBR, :]
    w1 = W_full[ANG + NBR:, :]
    p0, p1 = pl.pallas_call(
        _pre_body,
        grid=(GRID_E,),
        in_specs=[pl.BlockSpec((BLKE, NBR), lambda i: (i, 0)),
                  pl.BlockSpec((NBR, 2 * ANG), lambda i: (0, 0)),
                  pl.BlockSpec((NBR, 2 * ANG), lambda i: (0, 0))],
        out_specs=[pl.BlockSpec((BLKE, 2 * ANG), lambda i: (i, 0)),
                   pl.BlockSpec((BLKE, 2 * ANG), lambda i: (i, 0))],
        out_shape=[jax.ShapeDtypeStruct((E, 2 * ANG), f32),
                   jax.ShapeDtypeStruct((E, 2 * ANG), f32)],
    )(edge, w0, w1)

    idx0 = angle_nbr_idx[:, 0]
    idx1 = angle_nbr_idx[:, 1]
    g_sum = _sc_gather_add(p0, p1, idx0, idx1)

    idx3 = crystal_angle_idx.reshape(GRID, 1, BLK)

    angle_t = angle.T

    bounds = jnp.searchsorted(crystal_angle_idx,
                              jnp.arange(NC + 1, dtype=jnp.int32), side="left")
    cnt = jnp.maximum((bounds[1:] - bounds[:-1]).astype(f32), 1.0)

    row_spec = lambda w: pl.BlockSpec((BLK, w), lambda i: (i, 0))
    col_spec = lambda r: pl.BlockSpec((r, BLK), lambda i: (0, i))
    full_spec = lambda r, c: pl.BlockSpec((r, c), lambda i: (0, 0))
    idx_spec = pl.BlockSpec((1, 1, BLK), lambda i: (i, 0, 0))

    stats1 = pl.pallas_call(
        _p1_body,
        grid=(GRID,),
        in_specs=[col_spec(ANG), row_spec(2 * ANG),
                  full_spec(ANG, 2 * ANG), idx_spec],
        out_specs=full_spec(NC, 4 * ANG),
        out_shape=jax.ShapeDtypeStruct((NC, 4 * ANG), f32),
    )(angle_t, g_sum, wa, idx3)

    ab1 = _finalize(stats1, cnt, gamma1, beta1, 2 * ANG)
    wm = jnp.tile(W_mask, (1, ANG))

    eye = jnp.eye(ANG, dtype=f32)
    sumed_t, stats2 = pl.pallas_call(
        _p2_body,
        grid=(GRID,),
        in_specs=[col_spec(ANG), row_spec(2 * ANG),
                  full_spec(ANG, 2 * ANG), idx_spec,
                  full_spec(NC, 4 * ANG),
                  full_spec(ANG, ANG), full_spec(ANG, ANG)],
        out_specs=[col_spec(ANG), full_spec(NC, 2 * ANG)],
        out_shape=[jax.ShapeDtypeStruct((ANG, A), f32),
                   jax.ShapeDtypeStruct((NC, 2 * ANG), f32)],
    )(angle_t, g_sum, wa, idx3, ab1, wm, eye)

    ab2 = _finalize(stats2, cnt, gamma2, beta2, ANG)

    mid = ANG // 2
    w10 = jnp.zeros((ANG, 2 * ANG), f32).at[:, :mid].set(res_W1_0)
    b10 = jnp.zeros((2 * ANG, 1), f32).at[:mid, 0].set(res_b1_0)
    w20 = jnp.zeros((2 * ANG, ANG), f32).at[:mid, :].set(res_W2_0)
    b20 = res_b2_0.reshape(ANG, 1)
    w11 = jnp.zeros((ANG, 2 * ANG), f32).at[:, :mid].set(res_W1_1)
    b11 = jnp.zeros((2 * ANG, 1), f32).at[:mid, 0].set(res_b1_1)
    w21 = jnp.zeros((2 * ANG, ANG), f32).at[:mid, :].set(res_W2_1)
    b21 = res_b2_1.reshape(ANG, 1)

    out_t = pl.pallas_call(
        _p3_body,
        grid=(GRID,),
        in_specs=[col_spec(ANG), col_spec(ANG), idx_spec,
                  full_spec(NC, 2 * ANG),
                  full_spec(ANG, 2 * ANG), full_spec(2 * ANG, 1),
                  full_spec(2 * ANG, ANG), full_spec(ANG, 1),
                  full_spec(ANG, 2 * ANG), full_spec(2 * ANG, 1),
                  full_spec(2 * ANG, ANG), full_spec(ANG, 1)],
        out_specs=col_spec(ANG),
        out_shape=jax.ShapeDtypeStruct((ANG, A), f32),
    )(sumed_t, angle_t, idx3, ab2, w10, b10, w20, b20, w11, b11, w21, b21)

    return out_t.T
